# Initial kernel scaffold; baseline (speedup 1.0000x reference)
import jax
import jax.numpy as jnp
from jax.experimental import pallas as pl


def kernel(I, Ks, table):
    raise NotImplementedError("write your pallas kernel here")


# speedup vs baseline: 5.3252x; 5.3252x over previous
import functools

import jax
import jax.numpy as jnp
from jax import lax
from jax.experimental import pallas as pl
from jax.experimental.pallas import tpu as pltpu
from jax.experimental.pallas import tpu_sc as plsc

N_ITEMS = 100000
DIM = 128
BATCH = 4096
N_KS = 50

NW = 32
ROWS_PER_W = BATCH // NW
CHUNK_ROWS = 8
N_CHUNKS = ROWS_PER_W // CHUNK_ROWS
IDX_ROW = 100
GATHERS_PER_CHUNK = CHUNK_ROWS * N_KS // IDX_ROW

_LN2 = 0.6931471805599453


def _vlog(v):
    bits = lax.bitcast_convert_type(v, jnp.int32)
    e = lax.shift_right_arithmetic(bits, 23) - 127
    m = lax.bitcast_convert_type(
        jnp.bitwise_or(jnp.bitwise_and(bits, 0x007FFFFF), 0x3F800000),
        jnp.float32)
    big = m > 1.4142135
    m = jnp.where(big, m * 0.5, m)
    e = jnp.where(big, e + 1, e)
    z = (m - 1.0) / (m + 1.0)
    z2 = z * z
    p = z * (2.0 + z2 * (2.0 / 3.0 + z2 * (2.0 / 5.0 + z2 * (2.0 / 7.0 + z2 * (2.0 / 9.0)))))
    return p + e.astype(jnp.float32) * _LN2


def _vsqrt(t):
    bits = lax.bitcast_convert_type(t, jnp.int32)
    y = lax.bitcast_convert_type(0x5F3759DF - lax.shift_right_arithmetic(bits, 1),
                                 jnp.float32)
    for _ in range(3):
        y = y * (1.5 - 0.5 * t * y * y)
    return t * y


def _sc_body(i_hbm, ks_hbm, table_hbm, out_hbm,
             i_idx, ks_idx, ui_rows, rows_buf, out_buf, sem):
    w = lax.axis_index("s") * 2 + lax.axis_index("c")
    lane = lax.iota(jnp.int32, 16)
    neg0 = jnp.where(lane == 0, -1.0, 1.0)

    pltpu.sync_copy(i_hbm.at[pl.ds(w * ROWS_PER_W, ROWS_PER_W)], i_idx)
    pltpu.async_copy(table_hbm.at[i_idx], ui_rows, sem).wait()
    pltpu.sync_copy(
        ks_hbm.at[pl.ds(w * (ROWS_PER_W * N_KS // IDX_ROW),
                        ROWS_PER_W * N_KS // IDX_ROW)], ks_idx)

    def row_body(r, c):
        b = c * CHUNK_ROWS + r
        u = [ui_rows[b, pl.ds(j * 16, 16)] for j in range(DIM // 16)]
        u[0] = u[0] * neg0

        zeros16 = jnp.zeros((16,), jnp.float32)
        wins = [zeros16, zeros16, zeros16, zeros16]
        for k in range(N_KS):
            rowi = r * N_KS + k
            acc = u[0] * rows_buf[rowi, pl.ds(0, 16)]
            for j in range(1, DIM // 16):
                acc = acc + u[j] * rows_buf[rowi, pl.ds(j * 16, 16)]
            s = jnp.sum(acc)
            if k < 32:
                wi = k // 16
                wins[wi] = jnp.where(lane == k - 16 * wi, s, wins[wi])
            elif k < 48:
                wins[2] = jnp.where(lane == k - 32, s, wins[2])
            if k >= 34:
                wins[3] = jnp.where(lane == k - 34, s, wins[3])

        logds = []
        inv_sum = zeros16
        for wi in range(4):
            x = -wins[wi]
            d = x + _vsqrt(x * x - 1.0)
            logds.append(_vlog(d))
            inv = 1.0 / d
            if wi == 3:
                inv = jnp.where(lane >= 14, inv, 0.0)
            inv_sum = inv_sum + inv
        s = jnp.sum(inv_sum)
        ls = _vlog(jnp.broadcast_to(s, (16,)))
        out_buf[b, pl.ds(34, 16)] = -logds[3] - ls
        out_buf[b, pl.ds(0, 16)] = -logds[0] - ls
        out_buf[b, pl.ds(16, 16)] = -logds[1] - ls
        out_buf[b, pl.ds(32, 16)] = -logds[2] - ls
        return c

    def chunk_body(c, carry):
        for g in range(GATHERS_PER_CHUNK):
            pltpu.async_copy(table_hbm.at[ks_idx.at[c * GATHERS_PER_CHUNK + g]],
                             rows_buf.at[pl.ds(g * IDX_ROW, IDX_ROW)], sem).wait()
        lax.fori_loop(0, CHUNK_ROWS, row_body, c)
        return carry

    lax.fori_loop(0, N_CHUNKS, chunk_body, 0)
    pltpu.sync_copy(out_buf, out_hbm.at[pl.ds(w * ROWS_PER_W, ROWS_PER_W)])


@functools.partial(jax.jit, static_argnums=())
def kernel(I, Ks, table):
    ks_flat = Ks.reshape(BATCH * N_KS // IDX_ROW, IDX_ROW)
    run = pl.kernel(
        _sc_body,
        out_type=jax.ShapeDtypeStruct((BATCH, N_KS), jnp.float32),
        mesh=plsc.VectorSubcoreMesh(core_axis_name="c", subcore_axis_name="s"),
        compiler_params=pltpu.CompilerParams(needs_layout_passes=False),
        scratch_types=[
            pltpu.VMEM((ROWS_PER_W,), jnp.int32),
            pltpu.VMEM((ROWS_PER_W * N_KS // IDX_ROW, IDX_ROW), jnp.int32),
            pltpu.VMEM((ROWS_PER_W, DIM), jnp.float32),
            pltpu.VMEM((CHUNK_ROWS * N_KS, DIM), jnp.float32),
            pltpu.VMEM((ROWS_PER_W, N_KS), jnp.float32),
            pltpu.SemaphoreType.DMA,
        ],
    )
    return run(I, ks_flat, table)


# speedup vs baseline: 7.8245x; 1.4693x over previous
import functools

import jax
import jax.numpy as jnp
from jax import lax
from jax.experimental import pallas as pl
from jax.experimental.pallas import tpu as pltpu
from jax.experimental.pallas import tpu_sc as plsc

N_ITEMS = 100000
DIM = 128
BATCH = 4096
N_KS = 50

NW = 32
ROWS_PER_W = BATCH // NW
CHUNK_ROWS = 4
N_CHUNKS = ROWS_PER_W // CHUNK_ROWS
IDX_ROW = 100
GATHERS_PER_CHUNK = CHUNK_ROWS * N_KS // IDX_ROW

_LN2 = 0.6931471805599453


def _vlog(v):
    bits = lax.bitcast_convert_type(v, jnp.int32)
    e = lax.shift_right_arithmetic(bits, 23) - 127
    m = lax.bitcast_convert_type(
        jnp.bitwise_or(jnp.bitwise_and(bits, 0x007FFFFF), 0x3F800000),
        jnp.float32)
    big = m > 1.4142135
    m = jnp.where(big, m * 0.5, m)
    e = jnp.where(big, e + 1, e)
    z = (m - 1.0) / (m + 1.0)
    z2 = z * z
    p = z * (2.0 + z2 * (2.0 / 3.0 + z2 * (2.0 / 5.0 + z2 * (2.0 / 7.0 + z2 * (2.0 / 9.0)))))
    return p + e.astype(jnp.float32) * _LN2


def _vsqrt(t):
    bits = lax.bitcast_convert_type(t, jnp.int32)
    y = lax.bitcast_convert_type(0x5F3759DF - lax.shift_right_arithmetic(bits, 1),
                                 jnp.float32)
    for _ in range(3):
        y = y * (1.5 - 0.5 * t * y * y)
    return t * y


def _sc_body(i_hbm, ks_hbm, table_hbm, out_hbm,
             i_idx, ks_idx, ui_rows, rows0, rows1, out_buf, sem, sem0, sem1):
    w = lax.axis_index("s") * 2 + lax.axis_index("c")
    lane = lax.iota(jnp.int32, 16)
    neg0 = jnp.where(lane == 0, -1.0, 1.0)

    pltpu.sync_copy(i_hbm.at[pl.ds(w * ROWS_PER_W, ROWS_PER_W)], i_idx)
    pltpu.async_copy(table_hbm.at[i_idx], ui_rows, sem).wait()
    pltpu.sync_copy(
        ks_hbm.at[pl.ds(w * (ROWS_PER_W * N_KS // IDX_ROW),
                        ROWS_PER_W * N_KS // IDX_ROW)], ks_idx)

    def start_gathers(c, buf, s):
        for g in range(GATHERS_PER_CHUNK):
            pltpu.async_copy(table_hbm.at[ks_idx.at[c * GATHERS_PER_CHUNK + g]],
                             buf.at[pl.ds(g * IDX_ROW, IDX_ROW)], s)

    def drain(buf, s):
        pltpu.make_async_copy(table_hbm.at[pl.ds(0, CHUNK_ROWS * N_KS)], buf,
                              s).wait()

    def row_body(r, c, rows_buf):
        b = c * CHUNK_ROWS + r
        u = [ui_rows[b, pl.ds(j * 16, 16)] for j in range(DIM // 16)]
        u[0] = u[0] * neg0

        zeros16 = jnp.zeros((16,), jnp.float32)
        wins = [zeros16, zeros16, zeros16, zeros16]
        for k in range(N_KS):
            rowi = r * N_KS + k
            acc = u[0] * rows_buf[rowi, pl.ds(0, 16)]
            for j in range(1, DIM // 16):
                acc = acc + u[j] * rows_buf[rowi, pl.ds(j * 16, 16)]
            s = jnp.sum(acc)
            if k < 32:
                wi = k // 16
                wins[wi] = jnp.where(lane == k - 16 * wi, s, wins[wi])
            elif k < 48:
                wins[2] = jnp.where(lane == k - 32, s, wins[2])
            if k >= 34:
                wins[3] = jnp.where(lane == k - 34, s, wins[3])

        logds = []
        inv_sum = zeros16
        for wi in range(4):
            x = -wins[wi]
            d = x + _vsqrt(x * x - 1.0)
            logds.append(_vlog(d))
            inv = 1.0 / d
            if wi == 3:
                inv = jnp.where(lane >= 14, inv, 0.0)
            inv_sum = inv_sum + inv
        s = jnp.sum(inv_sum)
        ls = _vlog(jnp.broadcast_to(s, (16,)))
        out_buf[b, pl.ds(34, 16)] = -logds[3] - ls
        out_buf[b, pl.ds(0, 16)] = -logds[0] - ls
        out_buf[b, pl.ds(16, 16)] = -logds[1] - ls
        out_buf[b, pl.ds(32, 16)] = -logds[2] - ls
        return c

    def compute_chunk(c, rows_buf):
        lax.fori_loop(0, CHUNK_ROWS, lambda r, cc: row_body(r, cc, rows_buf), c)

    start_gathers(0, rows0, sem0)

    def pair_body(cc, carry):
        c0 = 2 * cc
        start_gathers(c0 + 1, rows1, sem1)
        drain(rows0, sem0)
        compute_chunk(c0, rows0)

        @pl.when(c0 + 2 < N_CHUNKS)
        def _():
            start_gathers(c0 + 2, rows0, sem0)

        drain(rows1, sem1)
        compute_chunk(c0 + 1, rows1)
        return carry

    lax.fori_loop(0, N_CHUNKS // 2, pair_body, 0)
    pltpu.sync_copy(out_buf, out_hbm.at[pl.ds(w * ROWS_PER_W, ROWS_PER_W)])


@functools.partial(jax.jit, static_argnums=())
def kernel(I, Ks, table):
    ks_flat = Ks.reshape(BATCH * N_KS // IDX_ROW, IDX_ROW)
    run = pl.kernel(
        _sc_body,
        out_type=jax.ShapeDtypeStruct((BATCH, N_KS), jnp.float32),
        mesh=plsc.VectorSubcoreMesh(core_axis_name="c", subcore_axis_name="s"),
        compiler_params=pltpu.CompilerParams(needs_layout_passes=False),
        scratch_types=[
            pltpu.VMEM((ROWS_PER_W,), jnp.int32),
            pltpu.VMEM((ROWS_PER_W * N_KS // IDX_ROW, IDX_ROW), jnp.int32),
            pltpu.VMEM((ROWS_PER_W, DIM), jnp.float32),
            pltpu.VMEM((CHUNK_ROWS * N_KS, DIM), jnp.float32),
            pltpu.VMEM((CHUNK_ROWS * N_KS, DIM), jnp.float32),
            pltpu.VMEM((ROWS_PER_W, N_KS), jnp.float32),
            pltpu.SemaphoreType.DMA,
            pltpu.SemaphoreType.DMA,
            pltpu.SemaphoreType.DMA,
        ],
    )
    return run(I, ks_flat, table)


# speedup vs baseline: 7.8565x; 1.0041x over previous
import functools

import jax
import jax.numpy as jnp
from jax import lax
from jax.experimental import pallas as pl
from jax.experimental.pallas import tpu as pltpu
from jax.experimental.pallas import tpu_sc as plsc

N_ITEMS = 100000
DIM = 128
BATCH = 4096
N_KS = 50

NW = 32
ROWS_PER_W = BATCH // NW
CHUNK_ROWS = 4
N_CHUNKS = ROWS_PER_W // CHUNK_ROWS
IDX_ROW = 100
GATHERS_PER_CHUNK = CHUNK_ROWS * N_KS // IDX_ROW

_LN2 = 0.6931471805599453


def _vlog(v):
    bits = lax.bitcast_convert_type(v, jnp.int32)
    e = lax.shift_right_arithmetic(bits, 23) - 127
    m = lax.bitcast_convert_type(
        jnp.bitwise_or(jnp.bitwise_and(bits, 0x007FFFFF), 0x3F800000),
        jnp.float32)
    big = m > 1.4142135
    m = jnp.where(big, m * 0.5, m)
    e = jnp.where(big, e + 1, e)
    z = (m - 1.0) / (m + 1.0)
    z2 = z * z
    p = z * (2.0 + z2 * (2.0 / 3.0 + z2 * (2.0 / 5.0 + z2 * (2.0 / 7.0 + z2 * (2.0 / 9.0)))))
    return p + e.astype(jnp.float32) * _LN2


def _vsqrt(t):
    bits = lax.bitcast_convert_type(t, jnp.int32)
    y = lax.bitcast_convert_type(0x5F3759DF - lax.shift_right_arithmetic(bits, 1),
                                 jnp.float32)
    for _ in range(3):
        y = y * (1.5 - 0.5 * t * y * y)
    return t * y


def _sc_body(i_hbm, ks_hbm, table_hbm, out_hbm,
             i_idx, ks_idx, ui_rows, rows0, rows1, out_buf, sem, sem0, sem1):
    w = lax.axis_index("s") * 2 + lax.axis_index("c")
    lane = lax.iota(jnp.int32, 16)
    neg0 = jnp.where(lane == 0, -1.0, 1.0)

    pltpu.sync_copy(i_hbm.at[pl.ds(w * ROWS_PER_W, ROWS_PER_W)], i_idx)
    ui_cp = pltpu.async_copy(table_hbm.at[i_idx], ui_rows, sem)
    pltpu.sync_copy(
        ks_hbm.at[pl.ds(w * (ROWS_PER_W * N_KS // IDX_ROW),
                        ROWS_PER_W * N_KS // IDX_ROW)], ks_idx)

    def start_gathers(c, buf, s):
        for g in range(GATHERS_PER_CHUNK):
            pltpu.async_copy(table_hbm.at[ks_idx.at[c * GATHERS_PER_CHUNK + g]],
                             buf.at[pl.ds(g * IDX_ROW, IDX_ROW)], s)

    def drain(buf, s):
        pltpu.make_async_copy(table_hbm.at[pl.ds(0, CHUNK_ROWS * N_KS)], buf,
                              s).wait()

    def row_body(r, c, rows_buf):
        b = c * CHUNK_ROWS + r
        u = [ui_rows[b, pl.ds(j * 16, 16)] for j in range(DIM // 16)]
        u[0] = u[0] * neg0

        zeros16 = jnp.zeros((16,), jnp.float32)
        wins = [zeros16, zeros16, zeros16, zeros16]
        for k in range(N_KS):
            rowi = r * N_KS + k
            acc = u[0] * rows_buf[rowi, pl.ds(0, 16)]
            for j in range(1, DIM // 16):
                acc = acc + u[j] * rows_buf[rowi, pl.ds(j * 16, 16)]
            s = jnp.sum(acc)
            if k < 32:
                wi = k // 16
                wins[wi] = jnp.where(lane == k - 16 * wi, s, wins[wi])
            elif k < 48:
                wins[2] = jnp.where(lane == k - 32, s, wins[2])
            if k >= 34:
                wins[3] = jnp.where(lane == k - 34, s, wins[3])

        logds = []
        inv_sum = zeros16
        for wi in range(4):
            x = -wins[wi]
            d = x + _vsqrt(x * x - 1.0)
            logds.append(_vlog(d))
            inv = 1.0 / d
            if wi == 3:
                inv = jnp.where(lane >= 14, inv, 0.0)
            inv_sum = inv_sum + inv
        s = jnp.sum(inv_sum)
        ls = _vlog(jnp.broadcast_to(s, (16,)))
        out_buf[b, pl.ds(34, 16)] = -logds[3] - ls
        out_buf[b, pl.ds(0, 16)] = -logds[0] - ls
        out_buf[b, pl.ds(16, 16)] = -logds[1] - ls
        out_buf[b, pl.ds(32, 16)] = -logds[2] - ls
        return c

    def compute_chunk(c, rows_buf):
        lax.fori_loop(0, CHUNK_ROWS, lambda r, cc: row_body(r, cc, rows_buf), c)

    start_gathers(0, rows0, sem0)
    ui_cp.wait()

    def pair_body(cc, carry):
        c0 = 2 * cc
        start_gathers(c0 + 1, rows1, sem1)
        drain(rows0, sem0)
        compute_chunk(c0, rows0)

        @pl.when(c0 + 2 < N_CHUNKS)
        def _():
            start_gathers(c0 + 2, rows0, sem0)

        drain(rows1, sem1)
        compute_chunk(c0 + 1, rows1)
        return carry

    lax.fori_loop(0, N_CHUNKS // 2, pair_body, 0)
    pltpu.sync_copy(out_buf, out_hbm.at[pl.ds(w * ROWS_PER_W, ROWS_PER_W)])


@functools.partial(jax.jit, static_argnums=())
def kernel(I, Ks, table):
    ks_flat = Ks.reshape(BATCH * N_KS // IDX_ROW, IDX_ROW)
    run = pl.kernel(
        _sc_body,
        out_type=jax.ShapeDtypeStruct((BATCH, N_KS), jnp.float32),
        mesh=plsc.VectorSubcoreMesh(core_axis_name="c", subcore_axis_name="s"),
        compiler_params=pltpu.CompilerParams(needs_layout_passes=False),
        scratch_types=[
            pltpu.VMEM((ROWS_PER_W,), jnp.int32),
            pltpu.VMEM((ROWS_PER_W * N_KS // IDX_ROW, IDX_ROW), jnp.int32),
            pltpu.VMEM((ROWS_PER_W, DIM), jnp.float32),
            pltpu.VMEM((CHUNK_ROWS * N_KS, DIM), jnp.float32),
            pltpu.VMEM((CHUNK_ROWS * N_KS, DIM), jnp.float32),
            pltpu.VMEM((ROWS_PER_W, N_KS), jnp.float32),
            pltpu.SemaphoreType.DMA,
            pltpu.SemaphoreType.DMA,
            pltpu.SemaphoreType.DMA,
        ],
    )
    return run(I, ks_flat, table)


# speedup vs baseline: 8.4972x; 1.0816x over previous
import functools

import jax
import jax.numpy as jnp
from jax import lax
from jax.experimental import pallas as pl
from jax.experimental.pallas import tpu as pltpu
from jax.experimental.pallas import tpu_sc as plsc

N_ITEMS = 100000
DIM = 128
BATCH = 4096
N_KS = 50

NW = 32
ROWS_PER_W = BATCH // NW
CHUNK_ROWS = 2
N_CHUNKS = ROWS_PER_W // CHUNK_ROWS
IDX_ROW = 100
GATHERS_PER_CHUNK = CHUNK_ROWS * N_KS // IDX_ROW

_LN2 = 0.6931471805599453


def _vlog(v):
    bits = lax.bitcast_convert_type(v, jnp.int32)
    e = lax.shift_right_arithmetic(bits, 23) - 127
    m = lax.bitcast_convert_type(
        jnp.bitwise_or(jnp.bitwise_and(bits, 0x007FFFFF), 0x3F800000),
        jnp.float32)
    big = m > 1.4142135
    m = jnp.where(big, m * 0.5, m)
    e = jnp.where(big, e + 1, e)
    z = (m - 1.0) / (m + 1.0)
    z2 = z * z
    p = z * (2.0 + z2 * (2.0 / 3.0 + z2 * (2.0 / 5.0 + z2 * (2.0 / 7.0))))
    return p + e.astype(jnp.float32) * _LN2


def _vsqrt(t):
    bits = lax.bitcast_convert_type(t, jnp.int32)
    y = lax.bitcast_convert_type(0x5F3759DF - lax.shift_right_arithmetic(bits, 1),
                                 jnp.float32)
    for _ in range(3):
        y = y * (1.5 - 0.5 * t * y * y)
    return t * y


def _sc_body(i_hbm, ks_hbm, table_hbm, out_hbm,
             i_idx, ks_idx, ui_rows, rows0, rows1, out_buf, sem, sem0, sem1):
    w = lax.axis_index("s") * 2 + lax.axis_index("c")
    lane = lax.iota(jnp.int32, 16)
    neg0 = jnp.where(lane == 0, -1.0, 1.0)

    pltpu.sync_copy(i_hbm.at[pl.ds(w * ROWS_PER_W, ROWS_PER_W)], i_idx)
    ui_cp = pltpu.async_copy(table_hbm.at[i_idx], ui_rows, sem)
    pltpu.sync_copy(
        ks_hbm.at[pl.ds(w * (ROWS_PER_W * N_KS // IDX_ROW),
                        ROWS_PER_W * N_KS // IDX_ROW)], ks_idx)

    def start_gathers(c, buf, s):
        for g in range(GATHERS_PER_CHUNK):
            pltpu.async_copy(table_hbm.at[ks_idx.at[c * GATHERS_PER_CHUNK + g]],
                             buf.at[pl.ds(g * IDX_ROW, IDX_ROW)], s)

    def drain(c, buf, s):
        for g in range(GATHERS_PER_CHUNK):
            pltpu.make_async_copy(
                table_hbm.at[ks_idx.at[c * GATHERS_PER_CHUNK + g]],
                buf.at[pl.ds(g * IDX_ROW, IDX_ROW)], s).wait()

    def row_body(r, c, rows_buf):
        b = c * CHUNK_ROWS + r
        u = [ui_rows[b, pl.ds(j * 16, 16)] for j in range(DIM // 16)]
        u[0] = u[0] * neg0

        zeros16 = jnp.zeros((16,), jnp.float32)
        wins = [zeros16, zeros16, zeros16, zeros16]
        for k in range(N_KS):
            rowi = r * N_KS + k
            acc = u[0] * rows_buf[rowi, pl.ds(0, 16)]
            for j in range(1, DIM // 16):
                acc = acc + u[j] * rows_buf[rowi, pl.ds(j * 16, 16)]
            s = jnp.sum(acc)
            if k < 32:
                wi = k // 16
                wins[wi] = jnp.where(lane == k - 16 * wi, s, wins[wi])
            elif k < 48:
                wins[2] = jnp.where(lane == k - 32, s, wins[2])
            if k >= 34:
                wins[3] = jnp.where(lane == k - 34, s, wins[3])

        logds = []
        inv_sum = zeros16
        for wi in range(4):
            x = -wins[wi]
            sq = _vsqrt(x * x - 1.0)
            logds.append(_vlog(x + sq))
            inv = x - sq
            if wi == 3:
                inv = jnp.where(lane >= 14, inv, 0.0)
            inv_sum = inv_sum + inv
        s = jnp.sum(inv_sum)
        ls = _vlog(jnp.broadcast_to(s, (16,)))
        out_buf[b, pl.ds(34, 16)] = -logds[3] - ls
        out_buf[b, pl.ds(0, 16)] = -logds[0] - ls
        out_buf[b, pl.ds(16, 16)] = -logds[1] - ls
        out_buf[b, pl.ds(32, 16)] = -logds[2] - ls
        return c

    def compute_chunk(c, rows_buf):
        lax.fori_loop(0, CHUNK_ROWS, lambda r, cc: row_body(r, cc, rows_buf), c)

    start_gathers(0, rows0, sem0)
    ui_cp.wait()

    def pair_body(cc, carry):
        c0 = 2 * cc
        start_gathers(c0 + 1, rows1, sem1)
        drain(c0, rows0, sem0)
        compute_chunk(c0, rows0)

        @pl.when(c0 + 2 < N_CHUNKS)
        def _():
            start_gathers(c0 + 2, rows0, sem0)

        drain(c0 + 1, rows1, sem1)
        compute_chunk(c0 + 1, rows1)
        return carry

    lax.fori_loop(0, N_CHUNKS // 2, pair_body, 0)
    pltpu.sync_copy(out_buf, out_hbm.at[pl.ds(w * ROWS_PER_W, ROWS_PER_W)])


@functools.partial(jax.jit, static_argnums=())
def kernel(I, Ks, table):
    ks_flat = Ks.reshape(BATCH * N_KS // IDX_ROW, IDX_ROW)
    run = pl.kernel(
        _sc_body,
        out_type=jax.ShapeDtypeStruct((BATCH, N_KS), jnp.float32),
        mesh=plsc.VectorSubcoreMesh(core_axis_name="c", subcore_axis_name="s"),
        compiler_params=pltpu.CompilerParams(needs_layout_passes=False),
        scratch_types=[
            pltpu.VMEM((ROWS_PER_W,), jnp.int32),
            pltpu.VMEM((ROWS_PER_W * N_KS // IDX_ROW, IDX_ROW), jnp.int32),
            pltpu.VMEM((ROWS_PER_W, DIM), jnp.float32),
            pltpu.VMEM((CHUNK_ROWS * N_KS, DIM), jnp.float32),
            pltpu.VMEM((CHUNK_ROWS * N_KS, DIM), jnp.float32),
            pltpu.VMEM((ROWS_PER_W, N_KS), jnp.float32),
            pltpu.SemaphoreType.DMA,
            pltpu.SemaphoreType.DMA,
            pltpu.SemaphoreType.DMA,
        ],
    )
    return run(I, ks_flat, table)
